# baseline (speedup 1.0000x reference)
import functools

import jax
import jax.numpy as jnp
from jax import lax
from jax.experimental import pallas as pl
from jax.experimental.pallas import tpu as pltpu
from jax.experimental.pallas import tpu_sc as plsc

NC = 2
NS = 16
NW = NC * NS
K = 128
Z_ROWS = 128


def _fill_f32(ref, rows, cols, value):
    v16 = jnp.full((16,), value, dtype=jnp.float32)

    def body(i, _):
        for j in range(cols // 16):
            ref[i, pl.ds(j * 16, 16)] = v16
        return 0

    lax.fori_loop(0, rows, body, 0)


def _fill_f32_1d(ref, num, value):
    v16 = jnp.full((16,), value, dtype=jnp.float32)

    def body(i, _):
        ref[pl.ds(i * 16, 16)] = v16
        return 0

    lax.fori_loop(0, num // 16, body, 0)


def _make_deg_kernel(n_pad, e_pad):
    ew = e_pad // NW
    n_chunks = ew // K
    stripe = n_pad // NS
    mesh = plsc.VectorSubcoreMesh(core_axis_name="c", subcore_axis_name="s")

    @functools.partial(
        pl.kernel,
        mesh=mesh,
        out_type=jax.ShapeDtypeStruct((NC, n_pad), jnp.float32),
        scratch_types=[
            pltpu.VMEM((ew,), jnp.int32),
            pltpu.VMEM((K,), jnp.float32),
            pltpu.VMEM((stripe,), jnp.float32),
            pltpu.VMEM_SHARED((n_pad,), jnp.float32),
        ],
    )
    def deg_kernel(dst_hbm, out_hbm, didx, ones, zbuf, acc):
        c = lax.axis_index("c")
        s = lax.axis_index("s")
        ebase = pl.multiple_of((c * NS + s) * ew, 8)
        pltpu.sync_copy(dst_hbm.at[pl.ds(ebase, ew)], didx)
        _fill_f32_1d(ones, K, 1.0)
        _fill_f32_1d(zbuf, stripe, 0.0)
        pltpu.sync_copy(zbuf, acc.at[pl.ds(s * stripe, stripe)])
        plsc.subcore_barrier()

        def body(j, _):
            sl = pl.ds(pl.multiple_of(j * K, 8), K)
            pltpu.sync_copy(ones, acc.at[didx.at[sl]], add=True)
            return 0

        lax.fori_loop(0, n_chunks, body, 0)
        plsc.subcore_barrier()
        pltpu.sync_copy(
            acc.at[pl.ds(s * stripe, stripe)],
            out_hbm.at[c, pl.ds(s * stripe, stripe)],
        )

    return deg_kernel


def _make_prop_kernel(n_pad, e_pad, d):
    ew = e_pad // NW
    n_chunks = ew // K
    n4 = n_chunks // 4
    stripe = n_pad // NS
    mesh = plsc.VectorSubcoreMesh(core_axis_name="c", subcore_axis_name="s")

    @functools.partial(
        pl.kernel,
        mesh=mesh,
        out_type=jax.ShapeDtypeStruct((NC, n_pad, d), jnp.float32),
        scratch_types=[
            pltpu.VMEM((2, K), jnp.int32),
            pltpu.VMEM((2, K), jnp.int32),
            pltpu.VMEM((2, K), jnp.int32),
            pltpu.VMEM((2, K), jnp.int32),
            pltpu.VMEM((K, d), jnp.float32),
            pltpu.VMEM((K, d), jnp.float32),
            pltpu.VMEM_SHARED((n_pad, d), jnp.float32),
            pltpu.SemaphoreType.DMA,
            pltpu.SemaphoreType.DMA,
            pltpu.SemaphoreType.DMA,
            pltpu.SemaphoreType.DMA,
            pltpu.SemaphoreType.DMA,
            pltpu.SemaphoreType.DMA,
            pltpu.SemaphoreType.DMA,
            pltpu.SemaphoreType.DMA,
        ],
    )
    def prop_kernel(table_hbm, sd_hbm, out_hbm,
                    iba0, ibb0, iba1, ibb1, rowsa, rowsb, acc,
                    semia0, semib0, semia1, semib1, semga, semgb,
                    semsa, semsb):
        c = lax.axis_index("c")
        s = lax.axis_index("s")
        wbase = (c * NS + s) * n_chunks

        _fill_f32(rowsa, Z_ROWS, d, 0.0)
        for t in range(stripe // Z_ROWS):
            pltpu.sync_copy(
                rowsa,
                acc.at[pl.ds(s * stripe + t * Z_ROWS, Z_ROWS)],
            )
        plsc.subcore_barrier()

        def idx_start(ib, semi, j):
            pltpu.async_copy(sd_hbm.at[wbase + j], ib, semi)

        def idx_wait(ib, semi, j):
            pltpu.make_async_copy(sd_hbm.at[wbase + j], ib, semi).wait()

        def gather_start(ib, rows, semg):
            pltpu.async_copy(table_hbm.at[ib.at[0]], rows, semg)

        def gather_wait(ib, rows, semg):
            pltpu.make_async_copy(table_hbm.at[ib.at[0]], rows, semg).wait()

        def scatter_start(ib, rows, sems):
            pltpu.async_copy(rows, acc.at[ib.at[1]], sems, add=True)

        def scatter_wait(ib, rows, sems):
            pltpu.make_async_copy(rows, acc.at[ib.at[1]], sems).wait()

        idx_start(iba0, semia0, 0)
        idx_start(ibb0, semib0, 1)
        idx_start(iba1, semia1, 2)
        idx_start(ibb1, semib1, 3)

        def quad(j0, prefetch):
            idx_wait(iba0, semia0, j0)
            gather_start(iba0, rowsa, semga)
            idx_wait(ibb0, semib0, j0 + 1)
            gather_start(ibb0, rowsb, semgb)
            gather_wait(iba0, rowsa, semga)
            scatter_start(iba0, rowsa, semsa)
            gather_wait(ibb0, rowsb, semgb)
            scatter_start(ibb0, rowsb, semsb)
            scatter_wait(iba0, rowsa, semsa)
            if prefetch:
                idx_start(iba0, semia0, j0 + 4)
            idx_wait(iba1, semia1, j0 + 2)
            gather_start(iba1, rowsa, semga)
            gather_wait(iba1, rowsa, semga)
            scatter_start(iba1, rowsa, semsa)
            scatter_wait(ibb0, rowsb, semsb)
            if prefetch:
                idx_start(ibb0, semib0, j0 + 5)
            idx_wait(ibb1, semib1, j0 + 3)
            gather_start(ibb1, rowsb, semgb)
            gather_wait(ibb1, rowsb, semgb)
            scatter_start(ibb1, rowsb, semsb)
            scatter_wait(iba1, rowsa, semsa)
            if prefetch:
                idx_start(iba1, semia1, j0 + 6)
            scatter_wait(ibb1, rowsb, semsb)
            if prefetch:
                idx_start(ibb1, semib1, j0 + 7)

        def body(t, _):
            quad(t * 4, True)
            return 0

        lax.fori_loop(0, n4 - 1, body, 0)
        quad(n_chunks - 4, False)

        plsc.subcore_barrier()
        pltpu.sync_copy(
            acc.at[pl.ds(s * stripe, stripe)],
            out_hbm.at[c, pl.ds(s * stripe, stripe)],
        )

    return prop_kernel


def _tc_prep(x_ref, deg0_ref, deg1_ref, xn_ref, dinv_ref):
    d = deg0_ref[...] + deg1_ref[...] + 1.0
    dinv = lax.rsqrt(d)
    xn_ref[...] = x_ref[...] * dinv
    dinv_ref[...] = dinv


def _tc_layer(g0_ref, g1_ref, yn_ref, dinv_ref, w_ref, b_ref, out_ref):
    dinv = dinv_ref[...]
    agg = dinv * (g0_ref[...] + g1_ref[...] + yn_ref[...])
    z = jnp.dot(agg, w_ref[...], preferred_element_type=jnp.float32) + b_ref[...]
    out_ref[...] = jax.nn.relu(z) * dinv


def _tc_final(g0_ref, g1_ref, yn_ref, dinv_ref, w3_ref, b_ref, out_ref, *,
              n_classes):
    agg = dinv_ref[...] * (g0_ref[...] + g1_ref[...] + yn_ref[...])
    z = jnp.dot(agg, w3_ref[...], preferred_element_type=jnp.float32) + b_ref[...]
    cols = lax.broadcasted_iota(jnp.int32, z.shape, 1)
    zm = jnp.where(cols < n_classes, z, -1e30)
    m = jnp.max(zm, axis=1, keepdims=True)
    lse = jnp.log(jnp.sum(jnp.exp(zm - m), axis=1, keepdims=True))
    out_ref[...] = zm - m - lse


def kernel(x, edge_index, W1, b1, W2, b2, W3, b3):
    n, d_in = x.shape
    e = edge_index.shape[1]
    hid = W1.shape[1]
    n_classes = W3.shape[1]
    d3 = ((n_classes + 7) // 8) * 8

    n_pad = ((n + NS * Z_ROWS - 1) // (NS * Z_ROWS)) * (NS * Z_ROWS)
    chunk_e = NW * K * 4
    e_pad = ((e + chunk_e - 1) // chunk_e) * chunk_e
    npad_extra = n_pad - n
    pad_e = e_pad - e

    src = edge_index[0]
    dst = edge_index[1]
    if pad_e:
        spread = max(npad_extra, 1)
        k = jnp.arange(pad_e, dtype=jnp.int32)
        pad_idx = n + (k % spread)
        src = jnp.concatenate([src, pad_idx])
        dst = jnp.concatenate([dst, pad_idx])

    sd = jnp.stack(
        [src.reshape(e_pad // K, K), dst.reshape(e_pad // K, K)], axis=1
    )

    x_pad = jnp.pad(x, ((0, npad_extra), (0, 0)))
    w3p = jnp.pad(W3, ((0, 0), (0, d3 - n_classes)))
    b3p = jnp.pad(b3, (0, d3 - n_classes)).reshape(1, d3)

    deg_k = _make_deg_kernel(n_pad, e_pad)
    prop_h = _make_prop_kernel(n_pad, e_pad, hid)

    degs = deg_k(dst)
    deg2d = degs.reshape(NC, n_pad, 1)

    f32 = jnp.float32
    B = 1024
    nb = n_pad // B

    def row_block(d):
        return pl.BlockSpec((B, d), lambda i: (i, 0))

    def full_block(shape):
        return pl.BlockSpec(shape, lambda i: (0,) * len(shape))

    xn, dinv1 = pl.pallas_call(
        _tc_prep,
        grid=(nb,),
        in_specs=[row_block(d_in), row_block(1), row_block(1)],
        out_specs=(row_block(d_in), row_block(1)),
        out_shape=(
            jax.ShapeDtypeStruct((n_pad, d_in), f32),
            jax.ShapeDtypeStruct((n_pad, 1), f32),
        ),
    )(x_pad, deg2d[0], deg2d[1])

    layer_call = pl.pallas_call(
        _tc_layer,
        grid=(nb,),
        in_specs=[row_block(hid), row_block(hid), row_block(hid),
                  row_block(1), full_block((hid, hid)), full_block((1, hid))],
        out_specs=row_block(hid),
        out_shape=jax.ShapeDtypeStruct((n_pad, hid), f32),
    )

    g1 = prop_h(xn, sd)
    y2n = layer_call(g1[0], g1[1], xn, dinv1, W1, b1.reshape(1, hid))

    g2 = prop_h(y2n, sd)
    y3n = layer_call(g2[0], g2[1], y2n, dinv1, W2, b2.reshape(1, hid))

    g3 = prop_h(y3n, sd)
    out = pl.pallas_call(
        functools.partial(_tc_final, n_classes=n_classes),
        grid=(nb,),
        in_specs=[row_block(hid), row_block(hid), row_block(hid),
                  row_block(1), full_block((hid, d3)), full_block((1, d3))],
        out_specs=row_block(d3),
        out_shape=jax.ShapeDtypeStruct((n_pad, d3), f32),
    )(g3[0], g3[1], y3n, dinv1, w3p, b3p)

    return out[:n, :n_classes]


# speedup vs baseline: 1.1516x; 1.1516x over previous
import functools

import jax
import jax.numpy as jnp
from jax import lax
from jax.experimental import pallas as pl
from jax.experimental.pallas import tpu as pltpu
from jax.experimental.pallas import tpu_sc as plsc

NC = 2
NS = 16
NW = NC * NS
K = 128
Z_ROWS = 128


def _fill_f32(ref, rows, cols, value):
    v16 = jnp.full((16,), value, dtype=jnp.float32)

    def body(i, _):
        for j in range(cols // 16):
            ref[i, pl.ds(j * 16, 16)] = v16
        return 0

    lax.fori_loop(0, rows, body, 0)


def _fill_f32_1d(ref, num, value):
    v16 = jnp.full((16,), value, dtype=jnp.float32)

    def body(i, _):
        ref[pl.ds(i * 16, 16)] = v16
        return 0

    lax.fori_loop(0, num // 16, body, 0)


def _make_deg_kernel(n_pad, e_pad):
    ew = e_pad // NW
    n_chunks = ew // K
    stripe = n_pad // NS
    mesh = plsc.VectorSubcoreMesh(core_axis_name="c", subcore_axis_name="s")

    @functools.partial(
        pl.kernel,
        mesh=mesh,
        out_type=jax.ShapeDtypeStruct((NC, n_pad), jnp.float32),
        scratch_types=[
            pltpu.VMEM((ew,), jnp.int32),
            pltpu.VMEM((K,), jnp.float32),
            pltpu.VMEM((stripe,), jnp.float32),
            pltpu.VMEM_SHARED((n_pad,), jnp.float32),
        ],
    )
    def deg_kernel(dst_hbm, out_hbm, didx, ones, zbuf, acc):
        c = lax.axis_index("c")
        s = lax.axis_index("s")
        ebase = pl.multiple_of((c * NS + s) * ew, 8)
        pltpu.sync_copy(dst_hbm.at[pl.ds(ebase, ew)], didx)
        _fill_f32_1d(ones, K, 1.0)
        _fill_f32_1d(zbuf, stripe, 0.0)
        pltpu.sync_copy(zbuf, acc.at[pl.ds(s * stripe, stripe)])
        plsc.subcore_barrier()

        def body(j, _):
            sl = pl.ds(pl.multiple_of(j * K, 8), K)
            pltpu.sync_copy(ones, acc.at[didx.at[sl]], add=True)
            return 0

        lax.fori_loop(0, n_chunks, body, 0)
        plsc.subcore_barrier()
        pltpu.sync_copy(
            acc.at[pl.ds(s * stripe, stripe)],
            out_hbm.at[c, pl.ds(s * stripe, stripe)],
        )

    return deg_kernel


def _make_prop_kernel(n_pad, e_pad, d):
    ew = e_pad // NW
    n_chunks = ew // K
    n4 = n_chunks // 4
    stripe = n_pad // NS
    mesh = plsc.VectorSubcoreMesh(core_axis_name="c", subcore_axis_name="s")

    @functools.partial(
        pl.kernel,
        mesh=mesh,
        out_type=jax.ShapeDtypeStruct((NC, n_pad, d), jnp.float32),
        scratch_types=[
            pltpu.VMEM((2, K), jnp.int32),
            pltpu.VMEM((2, K), jnp.int32),
            pltpu.VMEM((2, K), jnp.int32),
            pltpu.VMEM((2, K), jnp.int32),
            pltpu.VMEM((K, d), jnp.float32),
            pltpu.VMEM((K, d), jnp.float32),
            pltpu.VMEM_SHARED((n_pad, d), jnp.float32),
            pltpu.SemaphoreType.DMA,
            pltpu.SemaphoreType.DMA,
            pltpu.SemaphoreType.DMA,
            pltpu.SemaphoreType.DMA,
            pltpu.SemaphoreType.DMA,
            pltpu.SemaphoreType.DMA,
        ],
    )
    def prop_kernel(table_hbm, sd_hbm, out_hbm,
                    iba0, ibb0, iba1, ibb1, rowsa, rowsb, acc,
                    semia0, semib0, semia1, semib1, semga, semgb):
        c = lax.axis_index("c")
        s = lax.axis_index("s")
        wbase = (c * NS + s) * n_chunks

        _fill_f32(rowsa, Z_ROWS, d, 0.0)
        for t in range(stripe // Z_ROWS):
            pltpu.sync_copy(
                rowsa,
                acc.at[pl.ds(s * stripe + t * Z_ROWS, Z_ROWS)],
            )
        plsc.subcore_barrier()

        def idx_start(ib, semi, j):
            pltpu.async_copy(sd_hbm.at[wbase + j], ib, semi)

        def idx_wait(ib, semi, j):
            pltpu.make_async_copy(sd_hbm.at[wbase + j], ib, semi).wait()

        def gather_start(ib, rows, semg):
            pltpu.async_copy(table_hbm.at[ib.at[0]], rows, semg)

        def gather_wait(ib, rows, semg):
            pltpu.make_async_copy(table_hbm.at[ib.at[0]], rows, semg).wait()

        def scatter(ib, rows):
            pltpu.sync_copy(rows, acc.at[ib.at[1]], add=True)

        idx_start(iba0, semia0, 0)
        idx_start(ibb0, semib0, 1)
        idx_start(iba1, semia1, 2)
        idx_start(ibb1, semib1, 3)

        def quad(j0, prefetch):
            idx_wait(iba0, semia0, j0)
            gather_start(iba0, rowsa, semga)
            idx_wait(ibb0, semib0, j0 + 1)
            gather_start(ibb0, rowsb, semgb)
            gather_wait(iba0, rowsa, semga)
            scatter(iba0, rowsa)
            if prefetch:
                idx_start(iba0, semia0, j0 + 4)
            idx_wait(iba1, semia1, j0 + 2)
            gather_start(iba1, rowsa, semga)
            gather_wait(ibb0, rowsb, semgb)
            scatter(ibb0, rowsb)
            if prefetch:
                idx_start(ibb0, semib0, j0 + 5)
            idx_wait(ibb1, semib1, j0 + 3)
            gather_start(ibb1, rowsb, semgb)
            gather_wait(iba1, rowsa, semga)
            scatter(iba1, rowsa)
            if prefetch:
                idx_start(iba1, semia1, j0 + 6)
            gather_wait(ibb1, rowsb, semgb)
            scatter(ibb1, rowsb)
            if prefetch:
                idx_start(ibb1, semib1, j0 + 7)

        def body(t, _):
            quad(t * 4, True)
            return 0

        lax.fori_loop(0, n4 - 1, body, 0)
        quad(n_chunks - 4, False)

        plsc.subcore_barrier()
        pltpu.sync_copy(
            acc.at[pl.ds(s * stripe, stripe)],
            out_hbm.at[c, pl.ds(s * stripe, stripe)],
        )

    return prop_kernel


def _tc_prep(x_ref, deg0_ref, deg1_ref, xn_ref, dinv_ref):
    d = deg0_ref[...] + deg1_ref[...] + 1.0
    dinv = lax.rsqrt(d)
    xn_ref[...] = x_ref[...] * dinv
    dinv_ref[...] = dinv


def _tc_layer(g0_ref, g1_ref, yn_ref, dinv_ref, w_ref, b_ref, out_ref):
    dinv = dinv_ref[...]
    agg = dinv * (g0_ref[...] + g1_ref[...] + yn_ref[...])
    z = jnp.dot(agg, w_ref[...], preferred_element_type=jnp.float32) + b_ref[...]
    out_ref[...] = jax.nn.relu(z) * dinv


def _tc_final(g0_ref, g1_ref, yn_ref, dinv_ref, w3_ref, b_ref, out_ref, *,
              n_classes):
    agg = dinv_ref[...] * (g0_ref[...] + g1_ref[...] + yn_ref[...])
    z = jnp.dot(agg, w3_ref[...], preferred_element_type=jnp.float32) + b_ref[...]
    cols = lax.broadcasted_iota(jnp.int32, z.shape, 1)
    zm = jnp.where(cols < n_classes, z, -1e30)
    m = jnp.max(zm, axis=1, keepdims=True)
    lse = jnp.log(jnp.sum(jnp.exp(zm - m), axis=1, keepdims=True))
    out_ref[...] = zm - m - lse


def kernel(x, edge_index, W1, b1, W2, b2, W3, b3):
    n, d_in = x.shape
    e = edge_index.shape[1]
    hid = W1.shape[1]
    n_classes = W3.shape[1]
    d3 = ((n_classes + 7) // 8) * 8

    n_pad = ((n + NS * Z_ROWS - 1) // (NS * Z_ROWS)) * (NS * Z_ROWS)
    chunk_e = NW * K * 4
    e_pad = ((e + chunk_e - 1) // chunk_e) * chunk_e
    npad_extra = n_pad - n
    pad_e = e_pad - e

    src = edge_index[0]
    dst = edge_index[1]
    if pad_e:
        spread = max(npad_extra, 1)
        k = jnp.arange(pad_e, dtype=jnp.int32)
        pad_idx = n + (k % spread)
        src = jnp.concatenate([src, pad_idx])
        dst = jnp.concatenate([dst, pad_idx])

    sd = jnp.stack(
        [src.reshape(e_pad // K, K), dst.reshape(e_pad // K, K)], axis=1
    )

    x_pad = jnp.pad(x, ((0, npad_extra), (0, 0)))
    w3p = jnp.pad(W3, ((0, 0), (0, d3 - n_classes)))
    b3p = jnp.pad(b3, (0, d3 - n_classes)).reshape(1, d3)

    deg_k = _make_deg_kernel(n_pad, e_pad)
    prop_h = _make_prop_kernel(n_pad, e_pad, hid)

    degs = deg_k(dst)
    deg2d = degs.reshape(NC, n_pad, 1)

    f32 = jnp.float32
    xn, dinv1 = pl.pallas_call(
        _tc_prep,
        out_shape=(
            jax.ShapeDtypeStruct((n_pad, d_in), f32),
            jax.ShapeDtypeStruct((n_pad, 1), f32),
        ),
    )(x_pad, deg2d[0], deg2d[1])

    layer_call = pl.pallas_call(
        _tc_layer,
        out_shape=jax.ShapeDtypeStruct((n_pad, hid), f32),
    )

    g1 = prop_h(xn, sd)
    y2n = layer_call(g1[0], g1[1], xn, dinv1, W1, b1.reshape(1, hid))

    g2 = prop_h(y2n, sd)
    y3n = layer_call(g2[0], g2[1], y2n, dinv1, W2, b2.reshape(1, hid))

    g3 = prop_h(y3n, sd)
    out = pl.pallas_call(
        functools.partial(_tc_final, n_classes=n_classes),
        out_shape=jax.ShapeDtypeStruct((n_pad, d3), f32),
    )(g3[0], g3[1], y3n, dinv1, w3p, b3p)

    return out[:n, :n_classes]
